# Initial kernel scaffold; baseline (speedup 1.0000x reference)
import jax
import jax.numpy as jnp
from jax.experimental import pallas as pl


def kernel(x, pos, batch, ptr, params):
    raise NotImplementedError("write your pallas kernel here")


# speedup vs baseline: 3.1981x; 3.1981x over previous
import functools

import jax
import jax.numpy as jnp
from jax.experimental import pallas as pl

_K = 16
_DEC = 4


def _knn_kern(full_ref, rows_ref, out_ref, *, s, t, k):
    d = None
    for c in range(3):
        a = rows_ref[0, :, c:c + 1]
        b = full_ref[0, c:c + 1, :]
        diff = a - b
        d = diff * diff if d is None else d + diff * diff
    col = jax.lax.broadcasted_iota(jnp.int32, (t, s), 1)
    outs = []
    for j in range(k):
        m = jnp.min(d, axis=1, keepdims=True)
        idx = jnp.min(jnp.where(d == m, col, s), axis=1, keepdims=True)
        outs.append(idx)
        if j + 1 < k:
            d = jnp.where(col == idx, jnp.inf, d)
    out_ref[...] = jnp.concatenate(outs, axis=1).reshape(1, t, k)


def _knn(pos, s, k):
    n = pos.shape[0]
    b = n // s
    p = pos.reshape(b, s, 3)
    pt = p.transpose(0, 2, 1)
    t = 256 if s >= 1024 else s
    grid = (b, s // t)
    idx = pl.pallas_call(
        functools.partial(_knn_kern, s=s, t=t, k=k),
        grid=grid,
        in_specs=[
            pl.BlockSpec((1, 3, s), lambda i, j: (i, 0, 0)),
            pl.BlockSpec((1, t, 3), lambda i, j: (i, j, 0)),
        ],
        out_specs=pl.BlockSpec((1, t, k), lambda i, j: (i, j, 0)),
        out_shape=jax.ShapeDtypeStruct((b, s, k), jnp.int32),
    )(pt, p)
    idx = idx + (jnp.arange(b, dtype=jnp.int32) * s)[:, None, None]
    return idx.reshape(-1, k)


def _interp_kern(fullt_ref, rows_ref, x_ref, out_ref, *, sc, t):
    d = None
    for c in range(3):
        a = rows_ref[0, :, c:c + 1]
        b = fullt_ref[0, c:c + 1, :]
        diff = a - b
        d = diff * diff if d is None else d + diff * diff
    col = jax.lax.broadcasted_iota(jnp.int32, (t, sc), 1)
    m = jnp.min(d, axis=1, keepdims=True)
    idx = jnp.min(jnp.where(d == m, col, sc), axis=1, keepdims=True)
    onehot = (col == idx).astype(jnp.float32)
    out_ref[...] = jax.lax.dot(
        onehot, x_ref[0], precision=jax.lax.Precision.HIGHEST,
        preferred_element_type=jnp.float32)[None]


def _interp(x, pos, pos_skip, s, s_skip):
    b = pos.shape[0] // s
    c = x.shape[1]
    pt = pos.reshape(b, s, 3).transpose(0, 2, 1)
    pf = pos_skip.reshape(b, s_skip, 3)
    xc = x.reshape(b, s, c)
    t = min(s_skip, 512)
    grid = (b, s_skip // t)
    out = pl.pallas_call(
        functools.partial(_interp_kern, sc=s, t=t),
        grid=grid,
        in_specs=[
            pl.BlockSpec((1, 3, s), lambda i, j: (i, 0, 0)),
            pl.BlockSpec((1, t, 3), lambda i, j: (i, j, 0)),
            pl.BlockSpec((1, s, c), lambda i, j: (i, 0, 0)),
        ],
        out_specs=pl.BlockSpec((1, t, c), lambda i, j: (i, j, 0)),
        out_shape=jax.ShapeDtypeStruct((b, s_skip, c), jnp.float32),
    )(pt, pf, xc)
    return out.reshape(b * s_skip, c)


def _apply_mlp(layers, x, act=True):
    for layer in layers:
        h = x @ layer['lin']['W']
        if 'b' in layer['lin']:
            h = h + layer['lin']['b']
        if 'bn' in layer:
            shp = h.shape
            h2 = h.reshape(-1, shp[-1])
            mu = h2.mean(0)
            var = h2.var(0)
            h2 = (h2 - mu) / jnp.sqrt(var + 1e-5) * layer['bn']['gamma'] + layer['bn']['beta']
            h = h2.reshape(shp)
        if act:
            h = jax.nn.leaky_relu(h, 0.2)
        x = h
    return x


def _lfa(p, x, pos, nbr):
    x_j = x[nbr]
    pos_j = pos[nbr]
    pos_i = jnp.broadcast_to(pos[:, None, :], pos_j.shape)
    pos_diff = pos_j - pos_i
    sq = jnp.sum(pos_diff ** 2, -1, keepdims=True)
    dist = jnp.where(sq > 0, jnp.sqrt(jnp.where(sq > 0, sq, 1.0)), 0.0)
    rel = jnp.concatenate([pos_i, pos_j, pos_diff, dist], -1)
    enc = _apply_mlp(p['enc'], rel)
    local = jnp.concatenate([x_j, enc], -1)
    att = _apply_mlp(p['att'], local, act=False)
    scores = jax.nn.softmax(att, axis=1)
    agg = jnp.sum(scores * local, axis=1)
    return _apply_mlp(p['post'], agg)


def _drb(p, x, pos, s):
    nbr = _knn(pos, s, _K)
    sc = _apply_mlp(p['shortcut'], x, act=False)
    h = _apply_mlp(p['mlp1'], x)
    h = _lfa(p['lfa1'], h, pos, nbr)
    h = _lfa(p['lfa2'], h, pos, nbr)
    h = _apply_mlp(p['mlp2'], h, act=False)
    return jax.nn.leaky_relu(h + sc, 0.2)


def _decim_idx(seed, n, s, dec):
    b = n // s
    sd = s // dec
    parts = []
    for i in range(b):
        perm = jax.random.permutation(
            jax.random.fold_in(jax.random.key(100 + seed), i), s)[:sd]
        parts.append(perm + i * s)
    return jnp.concatenate(parts)


def kernel(x, pos, batch, ptr, params):
    s1 = pos.shape[0] // (ptr.shape[0] - 1)
    s2 = s1 // _DEC
    s3 = s2 // _DEC
    s4 = s3 // _DEC
    s5 = s4 // _DEC
    h0 = x @ params['fc0']['W'] + params['fc0']['b']
    x1 = _drb(params['block1'], h0, pos, s1)
    i1 = _decim_idx(1, x1.shape[0], s1, _DEC)
    x1d = x1[i1]; p1d = pos[i1]
    x2 = _drb(params['block2'], x1d, p1d, s2)
    i2 = _decim_idx(2, x2.shape[0], s2, _DEC)
    x2d = x2[i2]; p2d = p1d[i2]
    x3 = _drb(params['block3'], x2d, p2d, s3)
    i3 = _decim_idx(3, x3.shape[0], s3, _DEC)
    x3d = x3[i3]; p3d = p2d[i3]
    x4 = _drb(params['block4'], x3d, p3d, s4)
    i4 = _decim_idx(4, x4.shape[0], s4, _DEC)
    x4d = x4[i4]; p4d = p3d[i4]
    xs = _apply_mlp(params['mlp_summit'], x4d)
    u = _interp(xs, p4d, p3d, s5, s4)
    f4 = _apply_mlp(params['fp4'], jnp.concatenate([u, x3d], 1))
    u = _interp(f4, p3d, p2d, s4, s3)
    f3 = _apply_mlp(params['fp3'], jnp.concatenate([u, x2d], 1))
    u = _interp(f3, p2d, p1d, s3, s2)
    f2 = _apply_mlp(params['fp2'], jnp.concatenate([u, x1d], 1))
    u = _interp(f2, p1d, pos, s2, s1)
    f1 = _apply_mlp(params['fp1'], jnp.concatenate([u, x1], 1))
    h = _apply_mlp(params['mlp_classif'], f1)
    logits = h @ params['fc_classif']['W'] + params['fc_classif']['b']
    return jax.nn.log_softmax(logits, axis=-1)


# speedup vs baseline: 3.3226x; 1.0389x over previous
import functools

import jax
import jax.numpy as jnp
from jax.experimental import pallas as pl

_K = 16
_DEC = 4


def _knn_kern(full_ref, rows_ref, out_ref, *, s, t, k):
    d = None
    for c in range(3):
        a = rows_ref[0, :, c:c + 1]
        b = full_ref[0, c:c + 1, :]
        diff = a - b
        d = diff * diff if d is None else d + diff * diff
    col = jax.lax.broadcasted_iota(jnp.int32, (t, s), 1)
    outs = []
    for j in range(k):
        idx = jnp.argmin(d, axis=1, keepdims=True).astype(jnp.int32)
        outs.append(idx)
        if j + 1 < k:
            d = jnp.where(col == idx, jnp.inf, d)
    out_ref[...] = jnp.concatenate(outs, axis=1).reshape(1, t, k)


def _knn(pos, s, k):
    n = pos.shape[0]
    b = n // s
    p = pos.reshape(b, s, 3)
    pt = p.transpose(0, 2, 1)
    t = 512 if s >= 1024 else s
    grid = (b, s // t)
    idx = pl.pallas_call(
        functools.partial(_knn_kern, s=s, t=t, k=k),
        grid=grid,
        in_specs=[
            pl.BlockSpec((1, 3, s), lambda i, j: (i, 0, 0)),
            pl.BlockSpec((1, t, 3), lambda i, j: (i, j, 0)),
        ],
        out_specs=pl.BlockSpec((1, t, k), lambda i, j: (i, j, 0)),
        out_shape=jax.ShapeDtypeStruct((b, s, k), jnp.int32),
    )(pt, p)
    idx = idx + (jnp.arange(b, dtype=jnp.int32) * s)[:, None, None]
    return idx.reshape(-1, k)


def _interp_kern(fullt_ref, rows_ref, x_ref, out_ref, *, sc, t):
    d = None
    for c in range(3):
        a = rows_ref[0, :, c:c + 1]
        b = fullt_ref[0, c:c + 1, :]
        diff = a - b
        d = diff * diff if d is None else d + diff * diff
    col = jax.lax.broadcasted_iota(jnp.int32, (t, sc), 1)
    m = jnp.min(d, axis=1, keepdims=True)
    idx = jnp.min(jnp.where(d == m, col, sc), axis=1, keepdims=True)
    onehot = (col == idx).astype(jnp.float32)
    out_ref[...] = jax.lax.dot(
        onehot, x_ref[0], precision=jax.lax.Precision.HIGHEST,
        preferred_element_type=jnp.float32)[None]


def _interp(x, pos, pos_skip, s, s_skip):
    b = pos.shape[0] // s
    c = x.shape[1]
    pt = pos.reshape(b, s, 3).transpose(0, 2, 1)
    pf = pos_skip.reshape(b, s_skip, 3)
    xc = x.reshape(b, s, c)
    t = min(s_skip, 512)
    grid = (b, s_skip // t)
    out = pl.pallas_call(
        functools.partial(_interp_kern, sc=s, t=t),
        grid=grid,
        in_specs=[
            pl.BlockSpec((1, 3, s), lambda i, j: (i, 0, 0)),
            pl.BlockSpec((1, t, 3), lambda i, j: (i, j, 0)),
            pl.BlockSpec((1, s, c), lambda i, j: (i, 0, 0)),
        ],
        out_specs=pl.BlockSpec((1, t, c), lambda i, j: (i, j, 0)),
        out_shape=jax.ShapeDtypeStruct((b, s_skip, c), jnp.float32),
    )(pt, pf, xc)
    return out.reshape(b * s_skip, c)


def _apply_mlp(layers, x, act=True):
    for layer in layers:
        h = x @ layer['lin']['W']
        if 'b' in layer['lin']:
            h = h + layer['lin']['b']
        if 'bn' in layer:
            shp = h.shape
            h2 = h.reshape(-1, shp[-1])
            mu = h2.mean(0)
            var = h2.var(0)
            h2 = (h2 - mu) / jnp.sqrt(var + 1e-5) * layer['bn']['gamma'] + layer['bn']['beta']
            h = h2.reshape(shp)
        if act:
            h = jax.nn.leaky_relu(h, 0.2)
        x = h
    return x


def _lfa(p, x, pos, nbr):
    x_j = x[nbr]
    pos_j = pos[nbr]
    pos_i = jnp.broadcast_to(pos[:, None, :], pos_j.shape)
    pos_diff = pos_j - pos_i
    sq = jnp.sum(pos_diff ** 2, -1, keepdims=True)
    dist = jnp.where(sq > 0, jnp.sqrt(jnp.where(sq > 0, sq, 1.0)), 0.0)
    rel = jnp.concatenate([pos_i, pos_j, pos_diff, dist], -1)
    enc = _apply_mlp(p['enc'], rel)
    local = jnp.concatenate([x_j, enc], -1)
    att = _apply_mlp(p['att'], local, act=False)
    scores = jax.nn.softmax(att, axis=1)
    agg = jnp.sum(scores * local, axis=1)
    return _apply_mlp(p['post'], agg)


def _drb(p, x, pos, s):
    nbr = _knn(pos, s, _K)
    sc = _apply_mlp(p['shortcut'], x, act=False)
    h = _apply_mlp(p['mlp1'], x)
    h = _lfa(p['lfa1'], h, pos, nbr)
    h = _lfa(p['lfa2'], h, pos, nbr)
    h = _apply_mlp(p['mlp2'], h, act=False)
    return jax.nn.leaky_relu(h + sc, 0.2)


def _decim_idx(seed, n, s, dec):
    b = n // s
    sd = s // dec
    parts = []
    for i in range(b):
        perm = jax.random.permutation(
            jax.random.fold_in(jax.random.key(100 + seed), i), s)[:sd]
        parts.append(perm + i * s)
    return jnp.concatenate(parts)


def kernel(x, pos, batch, ptr, params):
    s1 = pos.shape[0] // (ptr.shape[0] - 1)
    s2 = s1 // _DEC
    s3 = s2 // _DEC
    s4 = s3 // _DEC
    s5 = s4 // _DEC
    h0 = x @ params['fc0']['W'] + params['fc0']['b']
    x1 = _drb(params['block1'], h0, pos, s1)
    i1 = _decim_idx(1, x1.shape[0], s1, _DEC)
    x1d = x1[i1]; p1d = pos[i1]
    x2 = _drb(params['block2'], x1d, p1d, s2)
    i2 = _decim_idx(2, x2.shape[0], s2, _DEC)
    x2d = x2[i2]; p2d = p1d[i2]
    x3 = _drb(params['block3'], x2d, p2d, s3)
    i3 = _decim_idx(3, x3.shape[0], s3, _DEC)
    x3d = x3[i3]; p3d = p2d[i3]
    x4 = _drb(params['block4'], x3d, p3d, s4)
    i4 = _decim_idx(4, x4.shape[0], s4, _DEC)
    x4d = x4[i4]; p4d = p3d[i4]
    xs = _apply_mlp(params['mlp_summit'], x4d)
    u = _interp(xs, p4d, p3d, s5, s4)
    f4 = _apply_mlp(params['fp4'], jnp.concatenate([u, x3d], 1))
    u = _interp(f4, p3d, p2d, s4, s3)
    f3 = _apply_mlp(params['fp3'], jnp.concatenate([u, x2d], 1))
    u = _interp(f3, p2d, p1d, s3, s2)
    f2 = _apply_mlp(params['fp2'], jnp.concatenate([u, x1d], 1))
    u = _interp(f2, p1d, pos, s2, s1)
    f1 = _apply_mlp(params['fp1'], jnp.concatenate([u, x1], 1))
    h = _apply_mlp(params['mlp_classif'], f1)
    logits = h @ params['fc_classif']['W'] + params['fc_classif']['b']
    return jax.nn.log_softmax(logits, axis=-1)
